# baseline (speedup 1.0000x reference)
import functools

import jax
import jax.numpy as jnp
from jax import lax
from jax.experimental import pallas as pl
from jax.experimental.pallas import tpu as pltpu
from jax.experimental.pallas import tpu_sc as plsc


def _img_mean_body(x_ref, o_ref):
    o_ref[...] = jnp.sum(x_ref[...], axis=2) * (1.0 / 196.0)


def _bcast_body(inv_ref, img_ref, snd_ref, mimg_ref, msnd_ref):
    mimg_ref[...] = img_ref[...][None, :, :]
    row = jnp.sum(snd_ref[...], axis=1, keepdims=True) * inv_ref[0]
    msnd_ref[...] = jnp.broadcast_to(row, msnd_ref.shape)


def _make_sc_seg_sums(N, C, n_seg, seg):
    info = plsc.get_sparse_core_info()
    nw = info.num_cores * info.num_subcores
    halves = nw // n_seg
    rows_per_w = N // nw
    R = 128
    nk = rows_per_w // R
    ng = C // 16
    mesh = plsc.VectorSubcoreMesh(core_axis_name="c", subcore_axis_name="s")

    @functools.partial(
        pl.kernel,
        out_type=jax.ShapeDtypeStruct((n_seg, halves, C), jnp.float32),
        mesh=mesh,
        scratch_types=[
            pltpu.VMEM((2, R, C), jnp.float32),
            pltpu.VMEM((C,), jnp.float32),
            pltpu.SemaphoreType.DMA,
            pltpu.SemaphoreType.DMA,
        ],
    )
    def seg_sums(z_hbm, out_hbm, buf, row_v, sem0, sem1):
        wid = lax.axis_index("s") * info.num_cores + lax.axis_index("c")
        base = wid * rows_per_w
        sems = (sem0, sem1)

        def copy(k):
            return pltpu.make_async_copy(
                z_hbm.at[pl.ds(base + k * R, R), :], buf.at[k % 2], sems[k % 2]
            )

        copy(0).start()
        accs = tuple(jnp.zeros((16,), jnp.float32) for _ in range(ng))
        for k in range(nk):
            if k + 1 < nk:
                copy(k + 1).start()
            copy(k).wait()
            slot = buf.at[k % 2]

            def body(r, a, slot=slot):
                return tuple(
                    a[c] + slot[r, c * 16:(c + 1) * 16] for c in range(ng)
                )

            accs = lax.fori_loop(0, R, body, accs)
        for c in range(ng):
            row_v[c * 16:(c + 1) * 16] = accs[c]
        pltpu.sync_copy(row_v, out_hbm.at[wid // halves, wid % halves])

    return seg_sums


def kernel(Z_img, Z_snd, snd_splits):
    B, C, H, W = Z_img.shape
    HW = H * W
    S = 2048
    N = Z_snd.shape[0]
    n_seg = N // S

    snd_part = _make_sc_seg_sums(N, C, n_seg, S)(Z_snd)

    Z_img_flat = Z_img.reshape(B, C, HW)
    img_mean = pl.pallas_call(
        _img_mean_body,
        grid=(B // 8,),
        in_specs=[pl.BlockSpec((8, C, HW), lambda i: (i, 0, 0))],
        out_specs=pl.BlockSpec((8, C), lambda i: (i, 0)),
        out_shape=jax.ShapeDtypeStruct((B, C), jnp.float32),
    )(Z_img_flat)

    inv = (1.0 / jnp.asarray(snd_splits).astype(jnp.float32)).reshape(1)
    M_img, M_snd = pl.pallas_call(
        _bcast_body,
        grid=(n_seg,),
        in_specs=[
            pl.BlockSpec(memory_space=pltpu.SMEM),
            pl.BlockSpec((B, C), lambda i: (0, 0)),
            pl.BlockSpec((1, 2, C), lambda i: (i, 0, 0)),
        ],
        out_specs=[
            pl.BlockSpec((1, B, C), lambda i: (i, 0, 0)),
            pl.BlockSpec((1, B, C), lambda i: (i, 0, 0)),
        ],
        out_shape=[
            jax.ShapeDtypeStruct((n_seg, B, C), jnp.float32),
            jax.ShapeDtypeStruct((n_seg, B, C), jnp.float32),
        ],
    )(inv, img_mean, snd_part)
    return (M_img, M_snd)
